# baseline (speedup 1.0000x reference)
import jax
import jax.numpy as jnp
from jax import lax
from jax.experimental import pallas as pl
from jax.experimental.pallas import tpu as pltpu
from jax.experimental.pallas import tpu_sc as plsc

_N = 10000
_E = 320000
_D = 128
_H = 64
_C = 10
_G = 64
_ALPHA = 0.1

_NC = 2
_NS = 16
_NW = _NC * _NS
_EPT = _E // _NW
_GRP = 125
_NGRP = _EPT // _GRP
_NBUF = 4
_NSG = _NGRP // _NBUF
_INIT_ROWS = 1000
_INIT_TILES = _N // _INIT_ROWS
_DW = 8


def _sc_mesh():
    return plsc.VectorSubcoreMesh(
        core_axis_name="c", subcore_axis_name="s",
        num_cores=_NC, num_subcores=_NS)


def _deg_body(dst_hbm, ones_hbm, zeros_hbm, out_hbm, idx_v, ones_v, acc_sh,
              sem_s):
    c = lax.axis_index("c")
    s = lax.axis_index("s")
    wid = c * _NS + s
    pltpu.sync_copy(dst_hbm.at[wid], idx_v)
    pltpu.sync_copy(ones_hbm, ones_v)

    @pl.when(s < _INIT_TILES)
    def _():
        pltpu.sync_copy(zeros_hbm.at[pl.ds(s * _INIT_ROWS, _INIT_ROWS)],
                        acc_sh.at[pl.ds(s * _INIT_ROWS, _INIT_ROWS)])

    plsc.subcore_barrier()

    descs = [pltpu.async_copy(ones_v, acc_sh.at[idx_v.at[j]], sem_s, add=True)
             for j in range(_NGRP)]
    for d in descs:
        d.wait()
    plsc.subcore_barrier()

    @pl.when(s < _INIT_TILES)
    def _():
        pltpu.sync_copy(acc_sh.at[pl.ds(s * _INIT_ROWS, _INIT_ROWS)],
                        out_hbm.at[c, pl.ds(s * _INIT_ROWS, _INIT_ROWS)])


def _make_deg_call():
    return pl.kernel(
        _deg_body,
        out_type=jax.ShapeDtypeStruct((_NC, _N, _DW), jnp.float32),
        mesh=_sc_mesh(),
        compiler_params=pltpu.CompilerParams(use_tc_tiling_on_sc=False),
        scratch_types=[
            pltpu.VMEM((_NGRP, _GRP), jnp.int32),
            pltpu.VMEM((_GRP, _DW), jnp.float32),
            pltpu.VMEM_SHARED((_N, _DW), jnp.float32),
            pltpu.SemaphoreType.DMA,
        ],
    )


def _step_body(src_hbm, dst_hbm, g_hbm, out_hbm,
               idxs_v, idxd_v, rows_v, acc_sh, sem_g, sem_g2, sem_s):
    c = lax.axis_index("c")
    s = lax.axis_index("s")
    wid = c * _NS + s
    pltpu.sync_copy(src_hbm.at[wid], idxs_v)
    pltpu.sync_copy(dst_hbm.at[wid], idxd_v)

    @pl.when(s < _INIT_TILES)
    def _():
        pltpu.sync_copy(g_hbm.at[pl.ds(s * _INIT_ROWS, _INIT_ROWS)],
                        acc_sh.at[pl.ds(s * _INIT_ROWS, _INIT_ROWS)])

    plsc.subcore_barrier()

    def _fire_gathers(sg, base, sem):
        for b in range(_NBUF):
            pltpu.async_copy(g_hbm.at[idxs_v.at[sg * _NBUF + b]],
                             rows_v.at[base + b], sem)

    def _wait_gathers(sg, base, sem):
        for b in range(_NBUF):
            pltpu.make_async_copy(g_hbm.at[idxs_v.at[sg * _NBUF + b]],
                                  rows_v.at[base + b], sem).wait()

    def _scatter(sg, base):
        sd = [pltpu.async_copy(rows_v.at[base + b],
                               acc_sh.at[idxd_v.at[sg * _NBUF + b]],
                               sem_s, add=True)
              for b in range(_NBUF)]
        for d in sd:
            d.wait()

    _fire_gathers(0, 0, sem_g)

    def pair(p, carry):
        sg_a = 2 * p
        sg_b = 2 * p + 1
        _wait_gathers(sg_a, 0, sem_g)
        _fire_gathers(sg_b, _NBUF, sem_g2)
        _scatter(sg_a, 0)
        _wait_gathers(sg_b, _NBUF, sem_g2)

        @pl.when(p < _NSG // 2 - 1)
        def _():
            _fire_gathers(sg_b + 1, 0, sem_g)

        _scatter(sg_b, _NBUF)
        return carry

    lax.fori_loop(0, _NSG // 2, pair, 0)
    plsc.subcore_barrier()

    @pl.when(s < _INIT_TILES)
    def _():
        pltpu.sync_copy(acc_sh.at[pl.ds(s * _INIT_ROWS, _INIT_ROWS)],
                        out_hbm.at[c, pl.ds(s * _INIT_ROWS, _INIT_ROWS)])


def _make_step_call():
    return pl.kernel(
        _step_body,
        out_type=jax.ShapeDtypeStruct((_NC, _N, _H), jnp.float32),
        mesh=_sc_mesh(),
        compiler_params=pltpu.CompilerParams(use_tc_tiling_on_sc=False),
        scratch_types=[
            pltpu.VMEM((_NGRP, _GRP), jnp.int32),
            pltpu.VMEM((_NGRP, _GRP), jnp.int32),
            pltpu.VMEM((2 * _NBUF, _GRP, _H), jnp.float32),
            pltpu.VMEM_SHARED((_N, _H), jnp.float32),
            pltpu.SemaphoreType.DMA,
            pltpu.SemaphoreType.DMA,
            pltpu.SemaphoreType.DMA,
        ],
    )


_NP = _N // 2


def _mlp_body(x2_ref, w1l_ref, w1r_ref, b1p_ref, w2bd_ref, b2p_ref,
              degp_ref, sel16_ref,
              h0_ref, g0_ref, dinvp_ref):
    x2 = x2_ref[...]
    xe = x2[:, 0:_D]
    xo = x2[:, _D:2 * _D]
    hp = lax.dot_general(xe, w1l_ref[...], (((1,), (0,)), ((), ())),
                         preferred_element_type=jnp.float32) \
        + lax.dot_general(xo, w1r_ref[...], (((1,), (0,)), ((), ())),
                          preferred_element_type=jnp.float32)
    hp = jnp.maximum(hp + b1p_ref[...], 0.0)
    hp = lax.dot_general(hp, w2bd_ref[...], (((1,), (0,)), ((), ())),
                         preferred_element_type=jnp.float32) + b2p_ref[...]
    dd = degp_ref[...]
    degsum = dd[0] + dd[1]
    pairdeg = lax.dot_general(degsum, sel16_ref[...],
                              (((1,), (0,)), ((), ())),
                              preferred_element_type=jnp.float32)
    dinvp = lax.rsqrt(pairdeg + 1.0)
    h0_ref[...] = hp
    g0_ref[...] = hp * dinvp
    dinvp_ref[...] = dinvp


def _mlp_call(x2, w1l, w1r, b1p, w2bd, b2p, degp2, sel16):
    nblk = 5
    blk = _NP // nblk
    row_spec = pl.BlockSpec((blk, 2 * _H), lambda i: (i, 0))
    full = pl.BlockSpec
    return pl.pallas_call(
        _mlp_body,
        grid=(nblk,),
        in_specs=[
            pl.BlockSpec((blk, 2 * _D), lambda i: (i, 0)),
            full((_D, 2 * _H), lambda i: (0, 0)),
            full((_D, 2 * _H), lambda i: (0, 0)),
            full((1, 2 * _H), lambda i: (0, 0)),
            full((2 * _H, 2 * _H), lambda i: (0, 0)),
            full((1, 2 * _H), lambda i: (0, 0)),
            pl.BlockSpec((2, blk, 2 * _DW), lambda i: (0, i, 0)),
            full((2 * _DW, 2 * _H), lambda i: (0, 0)),
        ],
        out_specs=(row_spec, row_spec, row_spec),
        out_shape=(
            jax.ShapeDtypeStruct((_NP, 2 * _H), jnp.float32),
            jax.ShapeDtypeStruct((_NP, 2 * _H), jnp.float32),
            jax.ShapeDtypeStruct((_NP, 2 * _H), jnp.float32),
        ),
    )(x2, w1l, w1r, b1p, w2bd, b2p, degp2, sel16)


def _blend_body(sp_ref, g_ref, h0_ref, dinvp_ref, gout_ref):
    sp = sp_ref[...]
    S = sp[0] + sp[1] - g_ref[...]
    dinv = dinvp_ref[...]
    h1 = (1.0 - _ALPHA) * (dinv * S) + _ALPHA * h0_ref[...]
    gout_ref[...] = dinv * h1


def _blend_call(sp, g, h0, dinvp):
    return pl.pallas_call(
        _blend_body,
        out_shape=jax.ShapeDtypeStruct((_NP, 2 * _H), jnp.float32),
    )(sp, g, h0, dinvp)


def _final_body(sp_ref, g_ref, h0_ref, dinvp_ref, be_ref, bo_ref,
                w3_ref, b3_ref, out_ref):
    sp = sp_ref[...]
    S = sp[0] + sp[1] - g_ref[...]
    dinv = dinvp_ref[...]
    h2p = (1.0 - _ALPHA) * (dinv * S) + _ALPHA * h0_ref[...]
    h2e = lax.slice(h2p, (0, 0), (_NP, _H))
    h2o = lax.slice(h2p, (0, _H), (_NP, 2 * _H))
    gr = lax.broadcasted_iota(jnp.int32, (_G, 1), 0)
    ohe = (be_ref[...] == gr).astype(jnp.float32)
    oho = (bo_ref[...] == gr).astype(jnp.float32)
    dn = (((1,), (0,)), ((), ()))
    ssum = lax.dot_general(ohe, h2e, dn,
                           preferred_element_type=jnp.float32) \
        + lax.dot_general(oho, h2o, dn,
                          preferred_element_type=jnp.float32)
    ones_col = jnp.ones((_NP, 1), jnp.float32)
    cnt = lax.dot_general(ohe, ones_col, dn,
                          preferred_element_type=jnp.float32) \
        + lax.dot_general(oho, ones_col, dn,
                          preferred_element_type=jnp.float32)
    mean = ssum / jnp.maximum(cnt, 1.0)
    out_ref[...] = lax.dot_general(mean, w3_ref[...],
                                   (((1,), (1,)), ((), ())),
                                   preferred_element_type=jnp.float32) \
        + b3_ref[...]


def _final_call(sp, g, h0, dinvp, b_even, b_odd, w3, b3r):
    return pl.pallas_call(
        _final_body,
        out_shape=jax.ShapeDtypeStruct((_G, _C), jnp.float32),
    )(sp, g, h0, dinvp, b_even, b_odd, w3, b3r)


@jax.jit
def kernel(x, edge_index, batch, W1, b1, W2, b2, W3, b3):
    src3 = edge_index[0].reshape(_NW, _NGRP, _GRP)
    dst3 = edge_index[1].reshape(_NW, _NGRP, _GRP)
    ones_grp = jnp.ones((_GRP, _DW), jnp.float32)
    zeros_deg = jnp.zeros((_N, _DW), jnp.float32)

    zpad = jnp.zeros((_D, _H), jnp.float32)
    w1l = jnp.concatenate([W1.T, zpad], axis=1)
    w1r = jnp.concatenate([zpad, W1.T], axis=1)
    zh = jnp.zeros((_H, _H), jnp.float32)
    w2bd = jnp.concatenate(
        [jnp.concatenate([W2.T, zh], axis=1),
         jnp.concatenate([zh, W2.T], axis=1)], axis=0)
    b1p = jnp.concatenate([b1, b1]).reshape(1, 2 * _H)
    b2p = jnp.concatenate([b2, b2]).reshape(1, 2 * _H)
    sel16 = jnp.zeros((2 * _DW, 2 * _H), jnp.float32)
    sel16 = sel16.at[0, 0:_H].set(1.0).at[_DW, _H:2 * _H].set(1.0)

    x2 = x.reshape(_NP, 2 * _D)
    bpair = batch.reshape(_NP, 2)
    b_even = bpair[:, 0].reshape(1, _NP)
    b_odd = bpair[:, 1].reshape(1, _NP)

    deg_call = _make_deg_call()
    step_call = _make_step_call()

    degp = deg_call(dst3, ones_grp, zeros_deg)
    degp2 = degp.reshape(2, _NP, 2 * _DW)

    h0p, g0p, dinvp = _mlp_call(x2, w1l, w1r, b1p, w2bd, b2p,
                                degp2, sel16)

    sp1 = step_call(src3, dst3, g0p.reshape(_N, _H))
    g1p = _blend_call(sp1.reshape(_NC, _NP, 2 * _H), g0p, h0p, dinvp)

    sp2 = step_call(src3, dst3, g1p.reshape(_N, _H))
    out = _final_call(sp2.reshape(_NC, _NP, 2 * _H), g1p, h0p, dinvp,
                      b_even, b_odd, W3, b3.reshape(1, _C))
    return out


# speedup vs baseline: 1.0490x; 1.0490x over previous
import jax
import jax.numpy as jnp
from jax import lax
from jax.experimental import pallas as pl
from jax.experimental.pallas import tpu as pltpu
from jax.experimental.pallas import tpu_sc as plsc

_N = 10000
_E = 320000
_D = 128
_H = 64
_C = 10
_G = 64
_ALPHA = 0.1

_NC = 2
_NS = 16
_NW = _NC * _NS
_EPT = _E // _NW
_GRP = 250
_NGRP = _EPT // _GRP
_NBUF = 2
_NSG = _NGRP // _NBUF
_INIT_ROWS = 1000
_INIT_TILES = _N // _INIT_ROWS
_DW = 8


def _sc_mesh():
    return plsc.VectorSubcoreMesh(
        core_axis_name="c", subcore_axis_name="s",
        num_cores=_NC, num_subcores=_NS)


def _deg_body(edge_hbm, ones_hbm, zeros_hbm, out_hbm, idx_v, ones_v, acc_sh,
              sem_s):
    c = lax.axis_index("c")
    s = lax.axis_index("s")
    wid = c * _NS + s
    pltpu.sync_copy(edge_hbm.at[1, wid], idx_v)
    pltpu.sync_copy(ones_hbm, ones_v)

    @pl.when(s < _INIT_TILES)
    def _():
        pltpu.sync_copy(zeros_hbm.at[pl.ds(s * _INIT_ROWS, _INIT_ROWS)],
                        acc_sh.at[pl.ds(s * _INIT_ROWS, _INIT_ROWS)])

    plsc.subcore_barrier()

    descs = [pltpu.async_copy(ones_v, acc_sh.at[idx_v.at[j]], sem_s, add=True)
             for j in range(_NGRP)]
    for d in descs:
        d.wait()
    plsc.subcore_barrier()

    @pl.when(s < _INIT_TILES)
    def _():
        pltpu.sync_copy(acc_sh.at[pl.ds(s * _INIT_ROWS, _INIT_ROWS)],
                        out_hbm.at[c, pl.ds(s * _INIT_ROWS, _INIT_ROWS)])


def _make_deg_call():
    return pl.kernel(
        _deg_body,
        out_type=jax.ShapeDtypeStruct((_NC, _N, _DW), jnp.float32),
        mesh=_sc_mesh(),
        compiler_params=pltpu.CompilerParams(use_tc_tiling_on_sc=False),
        scratch_types=[
            pltpu.VMEM((_NGRP, _GRP), jnp.int32),
            pltpu.VMEM((_GRP, _DW), jnp.float32),
            pltpu.VMEM_SHARED((_N, _DW), jnp.float32),
            pltpu.SemaphoreType.DMA,
        ],
    )


def _step_body(edge_hbm, g_hbm, out_hbm,
               idxs_v, idxd_v, rows_v, acc_sh, sem_g, sem_g2, sem_s):
    c = lax.axis_index("c")
    s = lax.axis_index("s")
    wid = c * _NS + s
    pltpu.sync_copy(edge_hbm.at[0, wid], idxs_v)
    pltpu.sync_copy(edge_hbm.at[1, wid], idxd_v)

    @pl.when(s < _INIT_TILES)
    def _():
        pltpu.sync_copy(g_hbm.at[pl.ds(s * _INIT_ROWS, _INIT_ROWS)],
                        acc_sh.at[pl.ds(s * _INIT_ROWS, _INIT_ROWS)])

    plsc.subcore_barrier()

    def _fire_gathers(sg, base, sem):
        for b in range(_NBUF):
            pltpu.async_copy(g_hbm.at[idxs_v.at[sg * _NBUF + b]],
                             rows_v.at[base + b], sem)

    def _wait_gathers(sg, base, sem):
        for b in range(_NBUF):
            pltpu.make_async_copy(g_hbm.at[idxs_v.at[sg * _NBUF + b]],
                                  rows_v.at[base + b], sem).wait()

    def _scatter(sg, base):
        sd = [pltpu.async_copy(rows_v.at[base + b],
                               acc_sh.at[idxd_v.at[sg * _NBUF + b]],
                               sem_s, add=True)
              for b in range(_NBUF)]
        for d in sd:
            d.wait()

    _fire_gathers(0, 0, sem_g)

    def pair(p, carry):
        sg_a = 2 * p
        sg_b = 2 * p + 1
        _wait_gathers(sg_a, 0, sem_g)
        _fire_gathers(sg_b, _NBUF, sem_g2)
        _scatter(sg_a, 0)
        _wait_gathers(sg_b, _NBUF, sem_g2)

        @pl.when(p < _NSG // 2 - 1)
        def _():
            _fire_gathers(sg_b + 1, 0, sem_g)

        _scatter(sg_b, _NBUF)
        return carry

    lax.fori_loop(0, _NSG // 2, pair, 0)
    plsc.subcore_barrier()

    @pl.when(s < _INIT_TILES)
    def _():
        pltpu.sync_copy(acc_sh.at[pl.ds(s * _INIT_ROWS, _INIT_ROWS)],
                        out_hbm.at[c, pl.ds(s * _INIT_ROWS, _INIT_ROWS)])


def _make_step_call():
    return pl.kernel(
        _step_body,
        out_type=jax.ShapeDtypeStruct((_NC, _N, _H), jnp.float32),
        mesh=_sc_mesh(),
        compiler_params=pltpu.CompilerParams(use_tc_tiling_on_sc=False),
        scratch_types=[
            pltpu.VMEM((_NGRP, _GRP), jnp.int32),
            pltpu.VMEM((_NGRP, _GRP), jnp.int32),
            pltpu.VMEM((2 * _NBUF, _GRP, _H), jnp.float32),
            pltpu.VMEM_SHARED((_N, _H), jnp.float32),
            pltpu.SemaphoreType.DMA,
            pltpu.SemaphoreType.DMA,
            pltpu.SemaphoreType.DMA,
        ],
    )


_NP = _N // 2


def _mlp_body(x2_ref, w1l_ref, w1r_ref, b1p_ref, w2bd_ref, b2p_ref,
              degp_ref, sel16_ref,
              h0_ref, g0_ref, dinvp_ref):
    x2 = x2_ref[...]
    xe = x2[:, 0:_D]
    xo = x2[:, _D:2 * _D]
    hp = lax.dot_general(xe, w1l_ref[...], (((1,), (0,)), ((), ())),
                         preferred_element_type=jnp.float32) \
        + lax.dot_general(xo, w1r_ref[...], (((1,), (0,)), ((), ())),
                          preferred_element_type=jnp.float32)
    hp = jnp.maximum(hp + b1p_ref[...], 0.0)
    hp = lax.dot_general(hp, w2bd_ref[...], (((1,), (0,)), ((), ())),
                         preferred_element_type=jnp.float32) + b2p_ref[...]
    dd = degp_ref[...]
    degsum = dd[0] + dd[1]
    pairdeg = lax.dot_general(degsum, sel16_ref[...],
                              (((1,), (0,)), ((), ())),
                              preferred_element_type=jnp.float32)
    dinvp = lax.rsqrt(pairdeg + 1.0)
    h0_ref[...] = hp
    g0_ref[...] = hp * dinvp
    dinvp_ref[...] = dinvp


def _mlp_call(x2, w1l, w1r, b1p, w2bd, b2p, degp2, sel16):
    nblk = 1
    blk = _NP // nblk
    row_spec = pl.BlockSpec((blk, 2 * _H), lambda i: (i, 0))
    full = pl.BlockSpec
    return pl.pallas_call(
        _mlp_body,
        grid=(nblk,),
        in_specs=[
            pl.BlockSpec((blk, 2 * _D), lambda i: (i, 0)),
            full((_D, 2 * _H), lambda i: (0, 0)),
            full((_D, 2 * _H), lambda i: (0, 0)),
            full((1, 2 * _H), lambda i: (0, 0)),
            full((2 * _H, 2 * _H), lambda i: (0, 0)),
            full((1, 2 * _H), lambda i: (0, 0)),
            pl.BlockSpec((2, blk, 2 * _DW), lambda i: (0, i, 0)),
            full((2 * _DW, 2 * _H), lambda i: (0, 0)),
        ],
        out_specs=(row_spec, row_spec, row_spec),
        out_shape=(
            jax.ShapeDtypeStruct((_NP, 2 * _H), jnp.float32),
            jax.ShapeDtypeStruct((_NP, 2 * _H), jnp.float32),
            jax.ShapeDtypeStruct((_NP, 2 * _H), jnp.float32),
        ),
    )(x2, w1l, w1r, b1p, w2bd, b2p, degp2, sel16)


def _blend_body(sp_ref, g_ref, h0_ref, dinvp_ref, gout_ref):
    sp = sp_ref[...]
    S = sp[0] + sp[1] - g_ref[...]
    dinv = dinvp_ref[...]
    h1 = (1.0 - _ALPHA) * (dinv * S) + _ALPHA * h0_ref[...]
    gout_ref[...] = dinv * h1


def _blend_call(sp, g, h0, dinvp):
    return pl.pallas_call(
        _blend_body,
        out_shape=jax.ShapeDtypeStruct((_NP, 2 * _H), jnp.float32),
    )(sp, g, h0, dinvp)


def _final_body(sp_ref, g_ref, h0_ref, dinvp_ref, be_ref, bo_ref,
                w3_ref, b3_ref, out_ref):
    sp = sp_ref[...]
    S = sp[0] + sp[1] - g_ref[...]
    dinv = dinvp_ref[...]
    h2p = (1.0 - _ALPHA) * (dinv * S) + _ALPHA * h0_ref[...]
    h2e = lax.slice(h2p, (0, 0), (_NP, _H))
    h2o = lax.slice(h2p, (0, _H), (_NP, 2 * _H))
    gr = lax.broadcasted_iota(jnp.int32, (_G, 1), 0)
    ohe = (be_ref[...] == gr).astype(jnp.float32)
    oho = (bo_ref[...] == gr).astype(jnp.float32)
    dn = (((1,), (0,)), ((), ()))
    ssum = lax.dot_general(ohe, h2e, dn,
                           preferred_element_type=jnp.float32) \
        + lax.dot_general(oho, h2o, dn,
                          preferred_element_type=jnp.float32)
    ones_col = jnp.ones((_NP, 1), jnp.float32)
    cnt = lax.dot_general(ohe, ones_col, dn,
                          preferred_element_type=jnp.float32) \
        + lax.dot_general(oho, ones_col, dn,
                          preferred_element_type=jnp.float32)
    mean = ssum / jnp.maximum(cnt, 1.0)
    out_ref[...] = lax.dot_general(mean, w3_ref[...],
                                   (((1,), (1,)), ((), ())),
                                   preferred_element_type=jnp.float32) \
        + b3_ref[...]


def _final_call(sp, g, h0, dinvp, b_even, b_odd, w3, b3r):
    return pl.pallas_call(
        _final_body,
        out_shape=jax.ShapeDtypeStruct((_G, _C), jnp.float32),
    )(sp, g, h0, dinvp, b_even, b_odd, w3, b3r)


@jax.jit
def kernel(x, edge_index, batch, W1, b1, W2, b2, W3, b3):
    edges4 = edge_index.reshape(2, _NW, _NGRP, _GRP)
    ones_grp = jnp.ones((_GRP, _DW), jnp.float32)
    zeros_deg = jnp.zeros((_N, _DW), jnp.float32)

    zpad = jnp.zeros((_D, _H), jnp.float32)
    w1l = jnp.concatenate([W1.T, zpad], axis=1)
    w1r = jnp.concatenate([zpad, W1.T], axis=1)
    zh = jnp.zeros((_H, _H), jnp.float32)
    w2bd = jnp.concatenate(
        [jnp.concatenate([W2.T, zh], axis=1),
         jnp.concatenate([zh, W2.T], axis=1)], axis=0)
    b1p = jnp.concatenate([b1, b1]).reshape(1, 2 * _H)
    b2p = jnp.concatenate([b2, b2]).reshape(1, 2 * _H)
    sel16 = jnp.zeros((2 * _DW, 2 * _H), jnp.float32)
    sel16 = sel16.at[0, 0:_H].set(1.0).at[_DW, _H:2 * _H].set(1.0)

    x2 = x.reshape(_NP, 2 * _D)
    bpair = batch.reshape(_NP, 2)
    b_even = bpair[:, 0].reshape(1, _NP)
    b_odd = bpair[:, 1].reshape(1, _NP)

    deg_call = _make_deg_call()
    step_call = _make_step_call()

    degp = deg_call(edges4, ones_grp, zeros_deg)
    degp2 = degp.reshape(2, _NP, 2 * _DW)

    h0p, g0p, dinvp = _mlp_call(x2, w1l, w1r, b1p, w2bd, b2p,
                                degp2, sel16)

    sp1 = step_call(edges4, g0p.reshape(_N, _H))
    g1p = _blend_call(sp1.reshape(_NC, _NP, 2 * _H), g0p, h0p, dinvp)

    sp2 = step_call(edges4, g1p.reshape(_N, _H))
    out = _final_call(sp2.reshape(_NC, _NP, 2 * _H), g1p, h0p, dinvp,
                      b_even, b_odd, W3, b3.reshape(1, _C))
    return out
